# Initial kernel scaffold; baseline (speedup 1.0000x reference)
import jax
import jax.numpy as jnp
from jax.experimental import pallas as pl


def kernel(cat_indices, numericals, emb_tables, norm_mean, norm_std, W1, b1, W2, b2):
    raise NotImplementedError("write your pallas kernel here")


# speedup vs baseline: 20.3871x; 20.3871x over previous
import functools

import jax
import jax.numpy as jnp
from jax import lax
from jax.experimental import pallas as pl
from jax.experimental.pallas import tpu as pltpu
from jax.experimental.pallas import tpu_sc as plsc

B = 16384
F = 26
V = 1000
D = 64
NUM = 13
H = 1024
FD = F * D
BF = B * F
NUMP = 128

NC, NS = 2, 16
NW = NC * NS
ROWS_W = BF // NW
CHUNK = 128
CH = ROWS_W // CHUNK

_sc_mesh = plsc.VectorSubcoreMesh(core_axis_name="c", subcore_axis_name="s")


@functools.partial(
    pl.kernel,
    out_type=jax.ShapeDtypeStruct((BF, D), jnp.float32),
    mesh=_sc_mesh,
    scratch_types=[
        pltpu.VMEM((CH, CHUNK), jnp.int32),
        pltpu.VMEM((CHUNK, D), jnp.float32),
        pltpu.SemaphoreType.DMA,
    ],
    compiler_params=pltpu.CompilerParams(use_tc_tiling_on_sc=False),
)
def _sc_gather(table_hbm, idx_hbm, out_hbm, idx_v, rows_v, sem):
    wid = lax.axis_index("s") * NC + lax.axis_index("c")
    pltpu.sync_copy(idx_hbm.at[pl.ds(wid * CH, CH)], idx_v)
    base_row = wid * ROWS_W

    def body(j, carry):
        pltpu.async_copy(table_hbm.at[idx_v.at[j]], rows_v, sem).wait()
        pltpu.sync_copy(rows_v, out_hbm.at[pl.ds(base_row + j * CHUNK, CHUNK)])
        return carry

    lax.fori_loop(0, CH, body, 0)


def _mlp_body(emb_ref, num_ref, mean_ref, std_ref, w1e_ref, w1n_ref,
              b1_ref, w2_ref, b2_ref, out_ref):
    num = (num_ref[...] - mean_ref[...]) / std_ref[...]
    acc = jnp.dot(emb_ref[...], w1e_ref[...], preferred_element_type=jnp.float32)
    acc = acc + jnp.dot(num, w1n_ref[...], preferred_element_type=jnp.float32)
    x = jnp.maximum(acc + b1_ref[...], 0.0)
    out_ref[...] = jnp.sum(x * w2_ref[...], axis=1, keepdims=True) + b2_ref[...]


BB = 512


def _mlp(emb, num_p, mean_p, std_p, w1e, w1n, b1r, w2r, b2r):
    grid = (B // BB,)
    return pl.pallas_call(
        _mlp_body,
        grid=grid,
        in_specs=[
            pl.BlockSpec((BB, FD), lambda i: (i, 0)),
            pl.BlockSpec((BB, NUMP), lambda i: (i, 0)),
            pl.BlockSpec((1, NUMP), lambda i: (0, 0)),
            pl.BlockSpec((1, NUMP), lambda i: (0, 0)),
            pl.BlockSpec((FD, H), lambda i: (0, 0)),
            pl.BlockSpec((NUMP, H), lambda i: (0, 0)),
            pl.BlockSpec((1, H), lambda i: (0, 0)),
            pl.BlockSpec((1, H), lambda i: (0, 0)),
            pl.BlockSpec((1, 1), lambda i: (0, 0)),
        ],
        out_specs=pl.BlockSpec((BB, 1), lambda i: (i, 0)),
        out_shape=jax.ShapeDtypeStruct((B, 1), jnp.float32),
    )(emb, num_p, mean_p, std_p, w1e, w1n, b1r, w2r, b2r)


def kernel(cat_indices, numericals, emb_tables, norm_mean, norm_std, W1, b1, W2, b2):
    flat_table = emb_tables.reshape(F * V, D)
    offs = (jnp.arange(F, dtype=jnp.int32) * V)[None, :]
    flat_idx = (cat_indices.astype(jnp.int32) + offs).reshape(NW * CH, CHUNK)
    emb = _sc_gather(flat_table, flat_idx).reshape(B, FD)

    num_p = jnp.pad(numericals, ((0, 0), (0, NUMP - NUM)))
    mean_p = jnp.pad(norm_mean, (0, NUMP - NUM)).reshape(1, NUMP)
    std_p = jnp.pad(norm_std, (0, NUMP - NUM), constant_values=1.0).reshape(1, NUMP)
    w1e = W1[:FD]
    w1n = jnp.pad(W1[FD:], ((0, NUMP - NUM), (0, 0)))
    return _mlp(emb, num_p, mean_p, std_p, w1e, w1n,
                b1.reshape(1, H), W2.reshape(1, H), b2.reshape(1, 1))
